# baseline (speedup 1.0000x reference)
import functools

import jax
import jax.numpy as jnp
from jax import lax
from jax.experimental import pallas as pl
from jax.experimental.pallas import tpu as pltpu
from jax.experimental.pallas import tpu_sc as plsc

TARGET = 5
TEST = 0.0
EPS = 1e-6

NUM_CORES = 2
NUM_SUBCORES = 16
LANES = 16
NW = NUM_CORES * NUM_SUBCORES

ROWS = 16384
COLS = 128
ROWS_PER_W = ROWS // NW
CHUNK = 128
NCHUNK = ROWS_PER_W // CHUNK


NSLOT = 3


def _body(c_hbm, d_hbm, u_hbm, z_hbm, out_hbm,
          c_v, d_v, z_v, u_v, ic_v, id_v, iz1_v, iz2_v, sem_in, sem_out):
    wid = lax.axis_index("s") * NUM_CORES + lax.axis_index("c")
    pltpu.sync_copy(z_hbm, z_v)

    pltpu.sync_copy(u_hbm.at[pl.ds(wid * ROWS_PER_W, ROWS_PER_W)], u_v)

    def issue_in(ch):
        s = ch % NSLOT
        base = wid * ROWS_PER_W + ch * CHUNK
        return [
            pltpu.async_copy(c_hbm.at[pl.ds(base, CHUNK)], c_v[s], sem_in[s]),
            pltpu.async_copy(d_hbm.at[pl.ds(base, CHUNK)], d_v[s], sem_in[s]),
        ]

    def issue_out(ch):
        s = ch % NSLOT
        return [
            pltpu.async_copy(c_v[s], out_hbm.at[ic_v[s]], sem_out[s]),
            pltpu.async_copy(d_v[s], out_hbm.at[id_v[s]], sem_out[s]),
            pltpu.async_copy(z_v, out_hbm.at[iz1_v[s]], sem_out[s]),
            pltpu.async_copy(z_v, out_hbm.at[iz2_v[s]], sem_out[s]),
        ]

    handles_in = [None] * NCHUNK
    handles_out = [None] * NCHUNK
    handles_in[0] = issue_in(0)
    for ch in range(NCHUNK):
        s = ch % NSLOT
        if ch + 1 < NCHUNK:
            if ch + 1 >= NSLOT:
                for h in handles_out[ch + 1 - NSLOT]:
                    h.wait()
            handles_in[ch + 1] = issue_in(ch + 1)
        for h in handles_in[ch]:
            h.wait()
        _compute_chunk(wid, ch, c_v[s], d_v[s], u_v,
                       ic_v[s], id_v[s], iz1_v[s], iz2_v[s])
        handles_out[ch] = issue_out(ch)
    for ch in range(max(0, NCHUNK - NSLOT), NCHUNK):
        for h in handles_out[ch]:
            h.wait()


def _compute_chunk(wid, ch, c_v, d_v, u_v, ic_v, id_v, iz1_v, iz2_v):
    base = wid * ROWS_PER_W + ch * CHUNK
    for g in range(CHUNK // LANES):
        if True:
            r0 = g * LANES
            rows = lax.iota(jnp.int32, LANES) + r0
            five = jnp.full((LANES,), TARGET, jnp.int32)
            tc = plsc.load_gather(c_v, [rows, five])
            td = plsc.load_gather(d_v, [rows, five])
            lo = tc - td
            hi = tc + td
            t = jnp.float32(TEST)
            denom = jnp.maximum(hi - lo, EPS)
            p = (t - lo) / denom
            p = jnp.where(hi <= t, 1.0, jnp.where(lo > t, 0.0, p))
            p = jnp.clip(p, 0.0, 1.0)
            left = u_v[pl.ds(ch * CHUNK + r0, LANES)] < p

            min_h = jnp.minimum(hi, t)
            max_l = jnp.maximum(lo, t)
            c5 = jnp.where(left, (lo + min_h) * 0.5, (max_l + hi) * 0.5)
            d5 = jnp.where(left, (min_h - lo) * 0.5, (hi - max_l) * 0.5)
            plsc.store_scatter(c_v, [rows, five], c5)
            plsc.store_scatter(d_v, [rows, five], d5)

            r_glob = rows + base
            vbase = ((r_glob >> 3) << 5) + (r_glob & 7)
            zero = jnp.int32(0)
            one = jnp.int32(1)
            two = jnp.int32(2)
            three = jnp.int32(3)
            ic_v[pl.ds(r0, LANES)] = vbase + (jnp.where(left, zero, two) << 3)
            id_v[pl.ds(r0, LANES)] = vbase + (jnp.where(left, one, three) << 3)
            iz1_v[pl.ds(r0, LANES)] = vbase + (jnp.where(left, two, zero) << 3)
            iz2_v[pl.ds(r0, LANES)] = vbase + (jnp.where(left, three, one) << 3)


@functools.partial(
    pl.kernel,
    out_type=jax.ShapeDtypeStruct((ROWS * 4, COLS), jnp.float32),
    mesh=plsc.VectorSubcoreMesh(core_axis_name="c", subcore_axis_name="s"),
    compiler_params=pltpu.CompilerParams(needs_layout_passes=False),
    scratch_types=[
        [pltpu.VMEM((CHUNK, COLS), jnp.float32)] * NSLOT,
        [pltpu.VMEM((CHUNK, COLS), jnp.float32)] * NSLOT,
        pltpu.VMEM((CHUNK, COLS), jnp.float32),
        pltpu.VMEM((ROWS_PER_W,), jnp.float32),
        [pltpu.VMEM((CHUNK,), jnp.int32)] * NSLOT,
        [pltpu.VMEM((CHUNK,), jnp.int32)] * NSLOT,
        [pltpu.VMEM((CHUNK,), jnp.int32)] * NSLOT,
        [pltpu.VMEM((CHUNK,), jnp.int32)] * NSLOT,
        [pltpu.SemaphoreType.DMA] * NSLOT,
        [pltpu.SemaphoreType.DMA] * NSLOT,
    ],
)
def _ifelse_sc(c_hbm, d_hbm, u_hbm, z_hbm, out_hbm,
               c_v, d_v, z_v, u_v, ic_v, id_v, iz1_v, iz2_v, sem_in, sem_out):
    _body(c_hbm, d_hbm, u_hbm, z_hbm, out_hbm,
          c_v, d_v, z_v, u_v, ic_v, id_v, iz1_v, iz2_v, sem_in, sem_out)


def kernel(c, delta):
    u = jax.random.uniform(
        jax.random.key(42), (ROWS, 1), dtype=jnp.float32).reshape(ROWS)
    z = jnp.zeros((CHUNK, COLS), jnp.float32)
    out = _ifelse_sc(c, delta, u, z)
    out = out.reshape(ROWS // 8, 4, 8, COLS)
    out = out.transpose(0, 2, 1, 3)
    return out.reshape(ROWS, COLS * 4)


# speedup vs baseline: 1.0379x; 1.0379x over previous
import functools

import jax
import jax.numpy as jnp
from jax import lax
from jax.experimental import pallas as pl
from jax.experimental.pallas import tpu as pltpu
from jax.experimental.pallas import tpu_sc as plsc

TARGET = 5
TEST = 0.0
EPS = 1e-6

NUM_CORES = 2
NUM_SUBCORES = 16
LANES = 16
NW = NUM_CORES * NUM_SUBCORES

ROWS = 16384
COLS = 128
ROWS_PER_W = ROWS // NW
CHUNK = 128
NCHUNK = ROWS_PER_W // CHUNK


NSLOT = 3


def _body(c_hbm, d_hbm, u_hbm, z_hbm, out_hbm,
          c_v, d_v, z_v, u_v, ic_v, id_v, iz1_v, iz2_v, sem_in, sem_out):
    wid = lax.axis_index("s") * NUM_CORES + lax.axis_index("c")

    def issue_in(ch):
        s = ch % NSLOT
        base = wid * ROWS_PER_W + ch * CHUNK
        return [
            pltpu.async_copy(c_hbm.at[pl.ds(base, CHUNK)], c_v[s], sem_in[s]),
            pltpu.async_copy(d_hbm.at[pl.ds(base, CHUNK)], d_v[s], sem_in[s]),
        ]

    def issue_out(ch):
        s = ch % NSLOT
        return [
            pltpu.async_copy(c_v[s], out_hbm.at[ic_v[s]], sem_out[s]),
            pltpu.async_copy(d_v[s], out_hbm.at[id_v[s]], sem_out[s]),
            pltpu.async_copy(z_v, out_hbm.at[iz1_v[s]], sem_out[s]),
            pltpu.async_copy(z_v, out_hbm.at[iz2_v[s]], sem_out[s]),
        ]

    handles_in = [None] * NCHUNK
    handles_out = [None] * NCHUNK
    handles_in[0] = issue_in(0)
    pltpu.sync_copy(z_hbm, z_v)
    pltpu.sync_copy(u_hbm.at[pl.ds(wid * ROWS_PER_W, ROWS_PER_W)], u_v)
    for ch in range(NCHUNK):
        s = ch % NSLOT
        if ch + 1 < NCHUNK:
            if ch + 1 >= NSLOT:
                for h in handles_out[ch + 1 - NSLOT]:
                    h.wait()
            handles_in[ch + 1] = issue_in(ch + 1)
        for h in handles_in[ch]:
            h.wait()
        _compute_chunk(wid, ch, c_v[s], d_v[s], u_v,
                       ic_v[s], id_v[s], iz1_v[s], iz2_v[s])
        handles_out[ch] = issue_out(ch)
    for ch in range(max(0, NCHUNK - NSLOT), NCHUNK):
        for h in handles_out[ch]:
            h.wait()


def _compute_chunk(wid, ch, c_v, d_v, u_v, ic_v, id_v, iz1_v, iz2_v):
    base = wid * ROWS_PER_W + ch * CHUNK
    for g in range(CHUNK // LANES):
        if True:
            r0 = g * LANES
            rows = lax.iota(jnp.int32, LANES) + r0
            five = jnp.full((LANES,), TARGET, jnp.int32)
            tc = plsc.load_gather(c_v, [rows, five])
            td = plsc.load_gather(d_v, [rows, five])
            lo = tc - td
            hi = tc + td
            t = jnp.float32(TEST)
            denom = jnp.maximum(hi - lo, EPS)
            p = (t - lo) / denom
            p = jnp.where(hi <= t, 1.0, jnp.where(lo > t, 0.0, p))
            p = jnp.clip(p, 0.0, 1.0)
            left = u_v[pl.ds(ch * CHUNK + r0, LANES)] < p

            min_h = jnp.minimum(hi, t)
            max_l = jnp.maximum(lo, t)
            c5 = jnp.where(left, (lo + min_h) * 0.5, (max_l + hi) * 0.5)
            d5 = jnp.where(left, (min_h - lo) * 0.5, (hi - max_l) * 0.5)
            plsc.store_scatter(c_v, [rows, five], c5)
            plsc.store_scatter(d_v, [rows, five], d5)

            r_glob = rows + base
            vbase = ((r_glob >> 3) << 5) + (r_glob & 7)
            zero = jnp.int32(0)
            one = jnp.int32(1)
            two = jnp.int32(2)
            three = jnp.int32(3)
            ic_v[pl.ds(r0, LANES)] = vbase + (jnp.where(left, zero, two) << 3)
            id_v[pl.ds(r0, LANES)] = vbase + (jnp.where(left, one, three) << 3)
            iz1_v[pl.ds(r0, LANES)] = vbase + (jnp.where(left, two, zero) << 3)
            iz2_v[pl.ds(r0, LANES)] = vbase + (jnp.where(left, three, one) << 3)


@functools.partial(
    pl.kernel,
    out_type=jax.ShapeDtypeStruct((ROWS * 4, COLS), jnp.float32),
    mesh=plsc.VectorSubcoreMesh(core_axis_name="c", subcore_axis_name="s"),
    compiler_params=pltpu.CompilerParams(needs_layout_passes=False),
    scratch_types=[
        [pltpu.VMEM((CHUNK, COLS), jnp.float32)] * NSLOT,
        [pltpu.VMEM((CHUNK, COLS), jnp.float32)] * NSLOT,
        pltpu.VMEM((CHUNK, COLS), jnp.float32),
        pltpu.VMEM((ROWS_PER_W,), jnp.float32),
        [pltpu.VMEM((CHUNK,), jnp.int32)] * NSLOT,
        [pltpu.VMEM((CHUNK,), jnp.int32)] * NSLOT,
        [pltpu.VMEM((CHUNK,), jnp.int32)] * NSLOT,
        [pltpu.VMEM((CHUNK,), jnp.int32)] * NSLOT,
        [pltpu.SemaphoreType.DMA] * NSLOT,
        [pltpu.SemaphoreType.DMA] * NSLOT,
    ],
)
def _ifelse_sc(c_hbm, d_hbm, u_hbm, z_hbm, out_hbm,
               c_v, d_v, z_v, u_v, ic_v, id_v, iz1_v, iz2_v, sem_in, sem_out):
    _body(c_hbm, d_hbm, u_hbm, z_hbm, out_hbm,
          c_v, d_v, z_v, u_v, ic_v, id_v, iz1_v, iz2_v, sem_in, sem_out)


def kernel(c, delta):
    u = jax.random.uniform(
        jax.random.key(42), (ROWS, 1), dtype=jnp.float32).reshape(ROWS)
    z = jnp.zeros((CHUNK, COLS), jnp.float32)
    out = _ifelse_sc(c, delta, u, z)
    out = out.reshape(ROWS // 8, 4, 8, COLS)
    out = out.transpose(0, 2, 1, 3)
    return out.reshape(ROWS, COLS * 4)


# speedup vs baseline: 1.0663x; 1.0274x over previous
import functools

import jax
import jax.numpy as jnp
from jax import lax
from jax.experimental import pallas as pl
from jax.experimental.pallas import tpu as pltpu
from jax.experimental.pallas import tpu_sc as plsc

TARGET = 5
TEST = 0.0
EPS = 1e-6

NUM_CORES = 2
NUM_SUBCORES = 16
LANES = 16
NW = NUM_CORES * NUM_SUBCORES

ROWS = 16384
COLS = 128
ROWS_PER_W = ROWS // NW
CHUNK = 64
NCHUNK = ROWS_PER_W // CHUNK


NSLOT = 6


def _body(c_hbm, d_hbm, u_hbm, z_hbm, out_hbm,
          c_v, d_v, z_v, u_v, ic_v, id_v, iz1_v, iz2_v, sem_in, sem_out):
    wid = lax.axis_index("s") * NUM_CORES + lax.axis_index("c")

    def issue_in(ch):
        s = ch % NSLOT
        base = wid * ROWS_PER_W + ch * CHUNK
        return [
            pltpu.async_copy(c_hbm.at[pl.ds(base, CHUNK)], c_v[s], sem_in[s]),
            pltpu.async_copy(d_hbm.at[pl.ds(base, CHUNK)], d_v[s], sem_in[s]),
        ]

    def issue_out(ch):
        s = ch % NSLOT
        return [
            pltpu.async_copy(c_v[s], out_hbm.at[ic_v[s]], sem_out[s]),
            pltpu.async_copy(d_v[s], out_hbm.at[id_v[s]], sem_out[s]),
            pltpu.async_copy(z_v, out_hbm.at[iz1_v[s]], sem_out[s]),
            pltpu.async_copy(z_v, out_hbm.at[iz2_v[s]], sem_out[s]),
        ]

    handles_in = [None] * NCHUNK
    handles_out = [None] * NCHUNK
    handles_in[0] = issue_in(0)
    pltpu.sync_copy(z_hbm, z_v)
    pltpu.sync_copy(u_hbm.at[pl.ds(wid * ROWS_PER_W, ROWS_PER_W)], u_v)
    for ch in range(NCHUNK):
        s = ch % NSLOT
        if ch + 1 < NCHUNK:
            if ch + 1 >= NSLOT:
                for h in handles_out[ch + 1 - NSLOT]:
                    h.wait()
            handles_in[ch + 1] = issue_in(ch + 1)
        for h in handles_in[ch]:
            h.wait()
        _compute_chunk(wid, ch, c_v[s], d_v[s], u_v,
                       ic_v[s], id_v[s], iz1_v[s], iz2_v[s])
        handles_out[ch] = issue_out(ch)
    for ch in range(max(0, NCHUNK - NSLOT), NCHUNK):
        for h in handles_out[ch]:
            h.wait()


def _compute_chunk(wid, ch, c_v, d_v, u_v, ic_v, id_v, iz1_v, iz2_v):
    base = wid * ROWS_PER_W + ch * CHUNK
    for g in range(CHUNK // LANES):
        if True:
            r0 = g * LANES
            rows = lax.iota(jnp.int32, LANES) + r0
            five = jnp.full((LANES,), TARGET, jnp.int32)
            tc = plsc.load_gather(c_v, [rows, five])
            td = plsc.load_gather(d_v, [rows, five])
            lo = tc - td
            hi = tc + td
            t = jnp.float32(TEST)
            denom = jnp.maximum(hi - lo, EPS)
            p = (t - lo) / denom
            p = jnp.where(hi <= t, 1.0, jnp.where(lo > t, 0.0, p))
            p = jnp.clip(p, 0.0, 1.0)
            left = u_v[pl.ds(ch * CHUNK + r0, LANES)] < p

            min_h = jnp.minimum(hi, t)
            max_l = jnp.maximum(lo, t)
            c5 = jnp.where(left, (lo + min_h) * 0.5, (max_l + hi) * 0.5)
            d5 = jnp.where(left, (min_h - lo) * 0.5, (hi - max_l) * 0.5)
            plsc.store_scatter(c_v, [rows, five], c5)
            plsc.store_scatter(d_v, [rows, five], d5)

            r_glob = rows + base
            vbase = ((r_glob >> 3) << 5) + (r_glob & 7)
            zero = jnp.int32(0)
            one = jnp.int32(1)
            two = jnp.int32(2)
            three = jnp.int32(3)
            ic_v[pl.ds(r0, LANES)] = vbase + (jnp.where(left, zero, two) << 3)
            id_v[pl.ds(r0, LANES)] = vbase + (jnp.where(left, one, three) << 3)
            iz1_v[pl.ds(r0, LANES)] = vbase + (jnp.where(left, two, zero) << 3)
            iz2_v[pl.ds(r0, LANES)] = vbase + (jnp.where(left, three, one) << 3)


@functools.partial(
    pl.kernel,
    out_type=jax.ShapeDtypeStruct((ROWS * 4, COLS), jnp.float32),
    mesh=plsc.VectorSubcoreMesh(core_axis_name="c", subcore_axis_name="s"),
    compiler_params=pltpu.CompilerParams(needs_layout_passes=False),
    scratch_types=[
        [pltpu.VMEM((CHUNK, COLS), jnp.float32)] * NSLOT,
        [pltpu.VMEM((CHUNK, COLS), jnp.float32)] * NSLOT,
        pltpu.VMEM((CHUNK, COLS), jnp.float32),
        pltpu.VMEM((ROWS_PER_W,), jnp.float32),
        [pltpu.VMEM((CHUNK,), jnp.int32)] * NSLOT,
        [pltpu.VMEM((CHUNK,), jnp.int32)] * NSLOT,
        [pltpu.VMEM((CHUNK,), jnp.int32)] * NSLOT,
        [pltpu.VMEM((CHUNK,), jnp.int32)] * NSLOT,
        [pltpu.SemaphoreType.DMA] * NSLOT,
        [pltpu.SemaphoreType.DMA] * NSLOT,
    ],
)
def _ifelse_sc(c_hbm, d_hbm, u_hbm, z_hbm, out_hbm,
               c_v, d_v, z_v, u_v, ic_v, id_v, iz1_v, iz2_v, sem_in, sem_out):
    _body(c_hbm, d_hbm, u_hbm, z_hbm, out_hbm,
          c_v, d_v, z_v, u_v, ic_v, id_v, iz1_v, iz2_v, sem_in, sem_out)


def kernel(c, delta):
    u = jax.random.uniform(
        jax.random.key(42), (ROWS, 1), dtype=jnp.float32).reshape(ROWS)
    z = jnp.zeros((CHUNK, COLS), jnp.float32)
    out = _ifelse_sc(c, delta, u, z)
    out = out.reshape(ROWS // 8, 4, 8, COLS)
    out = out.transpose(0, 2, 1, 3)
    return out.reshape(ROWS, COLS * 4)
